# baseline (speedup 1.0000x reference)
import functools

import jax
import jax.numpy as jnp
from jax import lax
from jax.experimental import pallas as pl
from jax.experimental.pallas import tpu as pltpu
from jax.experimental.pallas import tpu_sc as plsc

B, S, D, H, DH = 1, 2048, 1024, 16, 64
M = 8192
K = 16
NG = 512
GSZ = M // NG

BS_A = 256
BM_A = 2048

BS_C = 256
F_C = 1024


def _stage_a_body(x_ref, g_ref, b_ref, wq_ref, bq_ref, kt_ref,
                  q_ref, sc_ref, bm_ref, q_scr):
    j = pl.program_id(1)

    @pl.when(j == 0)
    def _():
        xb = x_ref[...]
        mu = jnp.mean(xb, axis=1, keepdims=True)
        xc = xb - mu
        var = jnp.mean(xc * xc, axis=1, keepdims=True)
        xn = xc * lax.rsqrt(var + 1e-5) * g_ref[...] + b_ref[...]
        q = jnp.dot(xn, wq_ref[...], preferred_element_type=jnp.float32)
        q = q + bq_ref[...]
        q_scr[...] = q
        q_ref[...] = q

    s = lax.dot_general(q_scr[...], kt_ref[...],
                        (((1,), (1,)), ((), ())),
                        preferred_element_type=jnp.float32)
    sc_ref[...] = s
    p = jnp.maximum(jnp.maximum(s[:, 0:512], s[:, 512:1024]),
                    jnp.maximum(s[:, 1024:1536], s[:, 1536:2048]))

    @pl.when(j == 0)
    def _():
        bm_ref[...] = p

    @pl.when(j > 0)
    def _():
        bm_ref[...] = jnp.maximum(bm_ref[...], p)


def _stage_a(x2d, ln1_g, ln1_b, w_attn, b_attn, kv2d):
    grid = (S // BS_A, M // BM_A)
    return pl.pallas_call(
        _stage_a_body,
        grid=grid,
        in_specs=[
            pl.BlockSpec((BS_A, D), lambda i, j: (i, 0)),
            pl.BlockSpec((1, D), lambda i, j: (0, 0)),
            pl.BlockSpec((1, D), lambda i, j: (0, 0)),
            pl.BlockSpec((D, D), lambda i, j: (0, 0)),
            pl.BlockSpec((1, D), lambda i, j: (0, 0)),
            pl.BlockSpec((BM_A, D), lambda i, j: (j, 0)),
        ],
        out_specs=[
            pl.BlockSpec((BS_A, D), lambda i, j: (i, 0)),
            pl.BlockSpec((BS_A, BM_A), lambda i, j: (i, j)),
            pl.BlockSpec((BS_A, NG), lambda i, j: (i, 0)),
        ],
        out_shape=[
            jax.ShapeDtypeStruct((S, D), jnp.float32),
            jax.ShapeDtypeStruct((S, M), jnp.float32),
            jax.ShapeDtypeStruct((S, NG), jnp.float32),
        ],
        scratch_shapes=[pltpu.VMEM((BS_A, D), jnp.float32)],
    )(x2d, ln1_g.reshape(1, D), ln1_b.reshape(1, D), w_attn,
      b_attn.reshape(1, 3 * D), kv2d)


def _gelu_tanh(t):
    return 0.5 * t * (1.0 + jnp.tanh(0.7978845608028654 *
                                     (t + 0.044715 * t * t * t)))


def _stage_c_body(a_ref, x_ref, wp_ref, bp_ref, g2_ref, b2_ref,
                  wfc_ref, bfc_ref, wfp_ref, bfp_ref, out_ref, h2_scr):
    j = pl.program_id(1)

    @pl.when(j == 0)
    def _():
        h = jnp.dot(a_ref[...], wp_ref[...],
                    preferred_element_type=jnp.float32)
        h = h + bp_ref[...] + x_ref[...]
        mu = jnp.mean(h, axis=1, keepdims=True)
        hc = h - mu
        var = jnp.mean(hc * hc, axis=1, keepdims=True)
        h2 = hc * lax.rsqrt(var + 1e-5) * g2_ref[...] + b2_ref[...]
        h2_scr[...] = h2
        out_ref[...] = h + bfp_ref[...]

    t = jnp.dot(h2_scr[...], wfc_ref[...],
                preferred_element_type=jnp.float32) + bfc_ref[...]
    t = _gelu_tanh(t)
    out_ref[...] += jnp.dot(t, wfp_ref[...],
                            preferred_element_type=jnp.float32)


def _stage_c(attn_t, x2d, wp_p, b_proj, ln2_g, ln2_b, w_fc, b_fc,
             w_fc_proj, b_fc_proj):
    grid = (S // BS_C, (4 * D) // F_C)
    return pl.pallas_call(
        _stage_c_body,
        grid=grid,
        in_specs=[
            pl.BlockSpec((BS_C, D), lambda i, j: (i, 0)),
            pl.BlockSpec((BS_C, D), lambda i, j: (i, 0)),
            pl.BlockSpec((D, D), lambda i, j: (0, 0)),
            pl.BlockSpec((1, D), lambda i, j: (0, 0)),
            pl.BlockSpec((1, D), lambda i, j: (0, 0)),
            pl.BlockSpec((1, D), lambda i, j: (0, 0)),
            pl.BlockSpec((D, F_C), lambda i, j: (0, j)),
            pl.BlockSpec((1, F_C), lambda i, j: (0, j)),
            pl.BlockSpec((F_C, D), lambda i, j: (j, 0)),
            pl.BlockSpec((1, D), lambda i, j: (0, 0)),
        ],
        out_specs=pl.BlockSpec((BS_C, D), lambda i, j: (i, 0)),
        out_shape=jax.ShapeDtypeStruct((S, D), jnp.float32),
        scratch_shapes=[pltpu.VMEM((BS_C, D), jnp.float32)],
    )(attn_t, x2d, wp_p, b_proj.reshape(1, D), ln2_g.reshape(1, D),
      ln2_b.reshape(1, D), w_fc, b_fc.reshape(1, 4 * D), w_fc_proj,
      b_fc_proj.reshape(1, D))


NEG = -3.0e38
NW = 32
PPW = S // NW


def _tree_add(xs):
    xs = list(xs)
    while len(xs) > 1:
        xs = [xs[i] + xs[i + 1] for i in range(0, len(xs) - 1, 2)] + \
             ([xs[-1]] if len(xs) % 2 else [])
    return xs[0]


def _tree_max(xs):
    xs = list(xs)
    while len(xs) > 1:
        xs = [jnp.maximum(xs[i], xs[i + 1]) for i in range(0, len(xs) - 1, 2)] + \
             ([xs[-1]] if len(xs) % 2 else [])
    return xs[0]


def _sc_middle(q_t, scores, bm, kv_t):
    mesh = plsc.VectorSubcoreMesh(core_axis_name="c", subcore_axis_name="s")

    @functools.partial(
        pl.kernel,
        out_type=jax.ShapeDtypeStruct((S, D), jnp.float32),
        mesh=mesh,
        compiler_params=pltpu.CompilerParams(needs_layout_passes=False),
        scratch_types=[
            pltpu.VMEM((1, NG), jnp.float32),
            pltpu.VMEM((1, NG), jnp.float32),
            pltpu.VMEM((1, M), jnp.float32),
            pltpu.VMEM((1, M), jnp.float32),
            pltpu.VMEM((1, D), jnp.float32),
            pltpu.VMEM((1, D), jnp.float32),
            pltpu.VMEM((K,), jnp.int32),
            pltpu.VMEM((K,), jnp.int32),
            pltpu.VMEM((K, 2 * D), jnp.float32),
            pltpu.VMEM((K, 2 * D), jnp.float32),
            pltpu.VMEM((1, D), jnp.float32),
            pltpu.SemaphoreType.DMA,
            pltpu.SemaphoreType.DMA,
            pltpu.SemaphoreType.DMA,
        ],
    )
    def sc_fn(q_hbm, sc_hbm, bm_hbm, kv_hbm, out_hbm,
              bm0, bm1, sc0, sc1, q0, q1, mid0, mid1, kv0, kv1,
              o_v, sem_r, sem_q, sem_kv):
        bm_v = (bm0, bm1)
        sc_v = (sc0, sc1)
        q_v = (q0, q1)
        mid_v = (mid0, mid1)
        kv_v = (kv0, kv1)
        wid = lax.axis_index("s") * 2 + lax.axis_index("c")
        base = wid * PPW
        lane = lax.iota(jnp.int32, 16)

        def rows_copies(p_i, slot):
            s_pos = base + jnp.minimum(p_i, PPW - 1)
            return (
                pltpu.make_async_copy(bm_hbm.at[pl.ds(s_pos, 1)],
                                      bm_v[slot], sem_r),
                pltpu.make_async_copy(sc_hbm.at[pl.ds(s_pos, 1)],
                                      sc_v[slot], sem_r),
                pltpu.make_async_copy(q_hbm.at[pl.ds(s_pos, 1)],
                                      q_v[slot], sem_q),
            )

        def kv_copy(slot):
            return pltpu.make_async_copy(kv_hbm.at[mid_v[slot]],
                                         kv_v[slot], sem_kv)

        def merge(tv, ti, c, p):
            cd, pd = plsc.sort_key_val(c, p, descending=True)
            m = tv >= cd
            nk = jnp.where(m, tv, cd)
            ni = jnp.where(m, ti, pd)
            return plsc.sort_key_val(nk, ni, descending=False)

        def topk_and_gather(p_i, slot):
            cb, cs, _ = rows_copies(p_i, slot)
            cb.wait()
            cs.wait()
            tv = jnp.full((16,), NEG, jnp.float32)
            ti = jnp.zeros((16,), jnp.int32)
            for g in range(NG // 16):
                tv, ti = merge(tv, ti, bm_v[slot][0, pl.ds(g * 16, 16)],
                               lane + g * 16)
            tv2 = jnp.full((16,), NEG, jnp.float32)
            ti2 = jnp.zeros((16,), jnp.int32)
            for r in range(GSZ):
                cidx = ti + r * NG
                c = plsc.load_gather(sc_v[slot], [jnp.zeros((16,), jnp.int32), cidx])
                tv2, ti2 = merge(tv2, ti2, c, cidx)
            mid_v[slot][...] = ti2
            kv_copy(slot).start()

        def attention(p_i, slot):
            _, _, cq = rows_copies(p_i, slot)
            cq.wait()
            kv_copy(slot).wait()

            lane_dh = lane * DH

            def qk_body(dd, accs):
                qd = plsc.load_gather(
                    q_v[slot], [jnp.zeros((16,), jnp.int32), lane_dh + dd])
                return tuple(accs[m] + qd * kv_v[slot][m, pl.ds(dd * 16, 16)]
                             for m in range(K))
            acc0 = tuple(jnp.zeros((16,), jnp.float32) for _ in range(K))
            accs = lax.fori_loop(0, DH, qk_body, acc0)
            sm = [a * (DH ** -0.5) for a in accs]
            mx = _tree_max(sm)
            es = [jnp.exp(a - mx) for a in sm]
            winv = 1.0 / _tree_add(es)
            ws = tuple(e * winv for e in es)

            def v_body(dd, c2):
                acc = _tree_add(
                    [ws[m] * kv_v[slot][m, pl.ds(D + dd * 16, 16)]
                     for m in range(K)])
                plsc.store_scatter(
                    o_v, [jnp.zeros((16,), jnp.int32), lane_dh + dd], acc)
                return c2
            lax.fori_loop(0, DH, v_body, 0)
            s_pos = base + p_i
            pltpu.sync_copy(o_v, out_hbm.at[pl.ds(s_pos, 1)])

        def issue_rows(p_i, slot):
            cb, cs, _ = rows_copies(p_i, slot)
            cb.start()
            cs.start()

        def issue_q(p_i, slot):
            _, _, cq = rows_copies(p_i, slot)
            cq.start()

        issue_rows(0, 0)
        issue_q(0, 0)

        def step(g, carry):
            for half in range(2):
                p_i = 2 * g + half
                slot = half
                other = 1 - half
                topk_and_gather(p_i, slot)
                issue_rows(p_i + 1, other)

                @pl.when(p_i >= 1)
                def _():
                    attention(p_i - 1, other)
                issue_q(p_i + 1, other)
            return carry

        lax.fori_loop(0, PPW // 2, step, 0)
        attention(PPW - 1, 1)
        cb, cs, cq = rows_copies(PPW, 0)
        cb.wait()
        cs.wait()
        cq.wait()

    return sc_fn(q_t, scores, bm, kv_t)


def _middle_jnp(q_t, scores, bm, kv_t):
    del bm
    _, idx = lax.top_k(scores, K)
    rows = jnp.take(kv_t, idx, axis=0)
    k_t = rows[..., :D].reshape(S, K, DH, H)
    v_t = rows[..., D:].reshape(S, K, DH, H)
    qh = q_t.reshape(S, DH, H)
    w = jnp.einsum('sdh,skdh->skh', qh, k_t) * (DH ** -0.5)
    w = jax.nn.softmax(w, axis=1)
    out = jnp.einsum('skh,skdh->sdh', w, v_t)
    return out.reshape(S, D)


def kernel(previous_hidden, ln1_g, ln1_b, W_attn, b_attn, W_proj, b_proj,
           ln2_g, ln2_b, W_fc, b_fc, W_fc_proj, b_fc_proj, mem_kv,
           attention_mask, head_mask):
    del attention_mask, head_mask
    x2d = previous_hidden.reshape(S, D)

    kv_t = mem_kv.reshape(M, 2, H, DH).transpose(0, 1, 3, 2).reshape(M, 2 * D)
    kv2d = mem_kv.reshape(M, 2 * D)

    q, scores, bm = _stage_a(x2d, ln1_g, ln1_b, W_attn, b_attn, kv2d)
    attn = _sc_middle(q, scores, bm, kv_t)
    out = _stage_c(attn, x2d, W_proj, b_proj, ln2_g, ln2_b, W_fc, b_fc,
                   W_fc_proj, b_fc_proj)
    return out.reshape(B, S, D)


# speedup vs baseline: 1.1587x; 1.1587x over previous
import functools

import jax
import jax.numpy as jnp
from jax import lax
from jax.experimental import pallas as pl
from jax.experimental.pallas import tpu as pltpu
from jax.experimental.pallas import tpu_sc as plsc

B, S, D, H, DH = 1, 2048, 1024, 16, 64
M = 8192
K = 16
NG = 512
GSZ = M // NG

BS_A = 256
BM_A = 2048

BS_C = 256
F_C = 1024


def _stage_a_body(x_ref, g_ref, b_ref, wq_ref, bq_ref, kt_ref,
                  q_ref, sc_ref, bm_ref, q_scr):
    j = pl.program_id(1)

    @pl.when(j == 0)
    def _():
        xb = x_ref[...]
        mu = jnp.mean(xb, axis=1, keepdims=True)
        xc = xb - mu
        var = jnp.mean(xc * xc, axis=1, keepdims=True)
        xn = xc * lax.rsqrt(var + 1e-5) * g_ref[...] + b_ref[...]
        q = jnp.dot(xn, wq_ref[...], preferred_element_type=jnp.float32)
        q = q + bq_ref[...]
        q_scr[...] = q
        q_ref[...] = q

    s = lax.dot_general(q_scr[...], kt_ref[...],
                        (((1,), (1,)), ((), ())),
                        preferred_element_type=jnp.float32)
    sc_ref[...] = s
    p = jnp.maximum(jnp.maximum(s[:, 0:512], s[:, 512:1024]),
                    jnp.maximum(s[:, 1024:1536], s[:, 1536:2048]))

    @pl.when(j == 0)
    def _():
        bm_ref[...] = p

    @pl.when(j > 0)
    def _():
        bm_ref[...] = jnp.maximum(bm_ref[...], p)


def _stage_a(x2d, ln1_g, ln1_b, wq_p, bq_p, kt, s_len):
    grid = (s_len // BS_A, M // BM_A)
    return pl.pallas_call(
        _stage_a_body,
        grid=grid,
        in_specs=[
            pl.BlockSpec((BS_A, D), lambda i, j: (i, 0)),
            pl.BlockSpec((1, D), lambda i, j: (0, 0)),
            pl.BlockSpec((1, D), lambda i, j: (0, 0)),
            pl.BlockSpec((D, D), lambda i, j: (0, 0)),
            pl.BlockSpec((1, D), lambda i, j: (0, 0)),
            pl.BlockSpec((BM_A, D), lambda i, j: (j, 0)),
        ],
        out_specs=[
            pl.BlockSpec((BS_A, D), lambda i, j: (i, 0)),
            pl.BlockSpec((BS_A, BM_A), lambda i, j: (i, j)),
            pl.BlockSpec((BS_A, NG), lambda i, j: (i, 0)),
        ],
        out_shape=[
            jax.ShapeDtypeStruct((s_len, D), jnp.float32),
            jax.ShapeDtypeStruct((s_len, M), jnp.float32),
            jax.ShapeDtypeStruct((s_len, NG), jnp.float32),
        ],
        scratch_shapes=[pltpu.VMEM((BS_A, D), jnp.float32)],
    )(x2d, ln1_g.reshape(1, D), ln1_b.reshape(1, D), wq_p,
      bq_p.reshape(1, D), kt)


def _gelu_tanh(t):
    return 0.5 * t * (1.0 + jnp.tanh(0.7978845608028654 *
                                     (t + 0.044715 * t * t * t)))


def _stage_c_body(a_ref, x_ref, wp_ref, bp_ref, g2_ref, b2_ref,
                  wfc_ref, bfc_ref, wfp_ref, bfp_ref, out_ref, h2_scr):
    j = pl.program_id(1)

    @pl.when(j == 0)
    def _():
        h = jnp.dot(a_ref[...], wp_ref[...],
                    preferred_element_type=jnp.float32)
        h = h + bp_ref[...] + x_ref[...]
        mu = jnp.mean(h, axis=1, keepdims=True)
        hc = h - mu
        var = jnp.mean(hc * hc, axis=1, keepdims=True)
        h2 = hc * lax.rsqrt(var + 1e-5) * g2_ref[...] + b2_ref[...]
        h2_scr[...] = h2
        out_ref[...] = h + bfp_ref[...]

    t = jnp.dot(h2_scr[...], wfc_ref[...],
                preferred_element_type=jnp.float32) + bfc_ref[...]
    t = _gelu_tanh(t)
    out_ref[...] += jnp.dot(t, wfp_ref[...],
                            preferred_element_type=jnp.float32)


def _stage_c(attn_t, x2d, wp_p, b_proj, ln2_g, ln2_b, w_fc, b_fc,
             w_fc_proj, b_fc_proj, s_len):
    grid = (s_len // BS_C, (4 * D) // F_C)
    return pl.pallas_call(
        _stage_c_body,
        grid=grid,
        in_specs=[
            pl.BlockSpec((BS_C, D), lambda i, j: (i, 0)),
            pl.BlockSpec((BS_C, D), lambda i, j: (i, 0)),
            pl.BlockSpec((D, D), lambda i, j: (0, 0)),
            pl.BlockSpec((1, D), lambda i, j: (0, 0)),
            pl.BlockSpec((1, D), lambda i, j: (0, 0)),
            pl.BlockSpec((1, D), lambda i, j: (0, 0)),
            pl.BlockSpec((D, F_C), lambda i, j: (0, j)),
            pl.BlockSpec((1, F_C), lambda i, j: (0, j)),
            pl.BlockSpec((F_C, D), lambda i, j: (j, 0)),
            pl.BlockSpec((1, D), lambda i, j: (0, 0)),
        ],
        out_specs=pl.BlockSpec((BS_C, D), lambda i, j: (i, 0)),
        out_shape=jax.ShapeDtypeStruct((s_len, D), jnp.float32),
        scratch_shapes=[pltpu.VMEM((BS_C, D), jnp.float32)],
    )(attn_t, x2d, wp_p, b_proj.reshape(1, D), ln2_g.reshape(1, D),
      ln2_b.reshape(1, D), w_fc, b_fc.reshape(1, 4 * D), w_fc_proj,
      b_fc_proj.reshape(1, D))


NEG = -3.0e38
NW = 32
PPW = S // NW


def _tree_add(xs):
    xs = list(xs)
    while len(xs) > 1:
        xs = [xs[i] + xs[i + 1] for i in range(0, len(xs) - 1, 2)] + \
             ([xs[-1]] if len(xs) % 2 else [])
    return xs[0]


def _tree_max(xs):
    xs = list(xs)
    while len(xs) > 1:
        xs = [jnp.maximum(xs[i], xs[i + 1]) for i in range(0, len(xs) - 1, 2)] + \
             ([xs[-1]] if len(xs) % 2 else [])
    return xs[0]


def _sc_middle(q_t, scores, bm, kv_t, s_len):
    mesh = plsc.VectorSubcoreMesh(core_axis_name="c", subcore_axis_name="s")

    @functools.partial(
        pl.kernel,
        out_type=jax.ShapeDtypeStruct((s_len, D), jnp.float32),
        mesh=mesh,
        compiler_params=pltpu.CompilerParams(needs_layout_passes=False),
        scratch_types=[
            pltpu.VMEM((1, NG), jnp.float32),
            pltpu.VMEM((1, NG), jnp.float32),
            pltpu.VMEM((1, M), jnp.float32),
            pltpu.VMEM((1, M), jnp.float32),
            pltpu.VMEM((1, D), jnp.float32),
            pltpu.VMEM((1, D), jnp.float32),
            pltpu.VMEM((K,), jnp.int32),
            pltpu.VMEM((K,), jnp.int32),
            pltpu.VMEM((K, 2 * D), jnp.float32),
            pltpu.VMEM((K, 2 * D), jnp.float32),
            pltpu.VMEM((1, D), jnp.float32),
            pltpu.SemaphoreType.DMA,
            pltpu.SemaphoreType.DMA,
            pltpu.SemaphoreType.DMA,
        ],
    )
    def sc_fn(q_hbm, sc_hbm, bm_hbm, kv_hbm, out_hbm,
              bm0, bm1, sc0, sc1, q0, q1, mid0, mid1, kv0, kv1,
              o_v, sem_r, sem_q, sem_kv):
        bm_v = (bm0, bm1)
        sc_v = (sc0, sc1)
        q_v = (q0, q1)
        mid_v = (mid0, mid1)
        kv_v = (kv0, kv1)
        ppw = s_len // NW
        wid = lax.axis_index("s") * 2 + lax.axis_index("c")
        base = wid * ppw
        lane = lax.iota(jnp.int32, 16)

        def rows_copies(p_i, slot):
            s_pos = base + jnp.minimum(p_i, ppw - 1)
            return (
                pltpu.make_async_copy(bm_hbm.at[pl.ds(s_pos, 1)],
                                      bm_v[slot], sem_r),
                pltpu.make_async_copy(sc_hbm.at[pl.ds(s_pos, 1)],
                                      sc_v[slot], sem_r),
                pltpu.make_async_copy(q_hbm.at[pl.ds(s_pos, 1)],
                                      q_v[slot], sem_q),
            )

        def kv_copy(slot):
            return pltpu.make_async_copy(kv_hbm.at[mid_v[slot]],
                                         kv_v[slot], sem_kv)

        def merge(tv, ti, c, p):
            cd, pd = plsc.sort_key_val(c, p, descending=True)
            m = tv >= cd
            nk = jnp.where(m, tv, cd)
            ni = jnp.where(m, ti, pd)
            return plsc.sort_key_val(nk, ni, descending=False)

        def topk_and_gather(p_i, slot):
            cb, cs, _ = rows_copies(p_i, slot)
            cb.wait()
            cs.wait()
            tv = jnp.full((16,), NEG, jnp.float32)
            ti = jnp.zeros((16,), jnp.int32)
            for g in range(NG // 16):
                tv, ti = merge(tv, ti, bm_v[slot][0, pl.ds(g * 16, 16)],
                               lane + g * 16)
            tv2 = jnp.full((16,), NEG, jnp.float32)
            ti2 = jnp.zeros((16,), jnp.int32)
            for r in range(GSZ):
                cidx = ti + r * NG
                c = plsc.load_gather(sc_v[slot], [jnp.zeros((16,), jnp.int32), cidx])
                tv2, ti2 = merge(tv2, ti2, c, cidx)
            mid_v[slot][...] = ti2
            kv_copy(slot).start()

        def attention(p_i, slot):
            _, _, cq = rows_copies(p_i, slot)
            cq.wait()
            kv_copy(slot).wait()

            def qk_body(dd, accs):
                qd = q_v[slot][0, pl.ds(dd * 16, 16)]
                return tuple(accs[m] + qd * kv_v[slot][m, pl.ds(dd * 16, 16)]
                             for m in range(K))
            acc0 = tuple(jnp.zeros((16,), jnp.float32) for _ in range(K))
            accs = lax.fori_loop(0, DH, qk_body, acc0)
            sm = [a * (DH ** -0.5) for a in accs]
            mx = _tree_max(sm)
            es = [jnp.exp(a - mx) for a in sm]
            winv = 1.0 / _tree_add(es)
            ws = tuple(e * winv for e in es)

            def v_body(dd, c2):
                o_v[0, pl.ds(dd * 16, 16)] = _tree_add(
                    [ws[m] * kv_v[slot][m, pl.ds(D + dd * 16, 16)]
                     for m in range(K)])
                return c2
            lax.fori_loop(0, DH, v_body, 0)
            s_pos = base + p_i
            pltpu.sync_copy(o_v, out_hbm.at[pl.ds(s_pos, 1)])

        def issue_rows(p_i, slot):
            cb, cs, _ = rows_copies(p_i, slot)
            cb.start()
            cs.start()

        def issue_q(p_i, slot):
            _, _, cq = rows_copies(p_i, slot)
            cq.start()

        issue_rows(0, 0)
        issue_q(0, 0)

        def step(g, carry):
            for half in range(2):
                p_i = 2 * g + half
                slot = half
                other = 1 - half
                topk_and_gather(p_i, slot)
                issue_rows(p_i + 1, other)

                @pl.when(p_i >= 1)
                def _():
                    attention(p_i - 1, other)
                issue_q(p_i + 1, other)
            return carry

        lax.fori_loop(0, ppw // 2, step, 0)
        attention(ppw - 1, 1)
        cb, cs, cq = rows_copies(ppw, 0)
        cb.wait()
        cs.wait()
        cq.wait()

    return sc_fn(q_t, scores, bm, kv_t)


def _middle_jnp(q_t, scores, bm, kv_t):
    del bm
    _, idx = lax.top_k(scores, K)
    rows = jnp.take(kv_t, idx, axis=0)
    k_t = rows[..., :D].reshape(S, K, DH, H)
    v_t = rows[..., D:].reshape(S, K, DH, H)
    qh = q_t.reshape(S, DH, H)
    w = jnp.einsum('sdh,skdh->skh', qh, k_t) * (DH ** -0.5)
    w = jax.nn.softmax(w, axis=1)
    out = jnp.einsum('skh,skdh->sdh', w, v_t)
    return out.reshape(S, D)


def kernel(previous_hidden, ln1_g, ln1_b, W_attn, b_attn, W_proj, b_proj,
           ln2_g, ln2_b, W_fc, b_fc, W_fc_proj, b_fc_proj, mem_kv,
           attention_mask, head_mask):
    del attention_mask, head_mask
    x2d = previous_hidden.reshape(S, D)

    n = jnp.arange(D)
    perm = (n % H) * DH + n // H
    wq_p = W_attn[:, :D][:, perm]
    bq_p = b_attn[:D][perm]
    wp_p = W_proj[perm, :]
    kv_t = mem_kv.reshape(M, 2, H, DH).transpose(0, 1, 3, 2).reshape(M, 2 * D)

    n_chunks = 4
    s_len = S // n_chunks
    outs = []
    for c in range(n_chunks):
        xc = x2d[c * s_len:(c + 1) * s_len]
        q_t, scores, bm = _stage_a(xc, ln1_g, ln1_b, wq_p, bq_p,
                                   kv_t[:, :D], s_len)
        attn_t = _sc_middle(q_t, scores, bm, kv_t, s_len)
        outs.append(_stage_c(attn_t, xc, wp_p, b_proj, ln2_g, ln2_b, W_fc,
                             b_fc, W_fc_proj, b_fc_proj, s_len))
    out = jnp.concatenate(outs, axis=0)
    return out.reshape(B, S, D)
